# baseline (speedup 1.0000x reference)
import jax
import jax.numpy as jnp
from jax.experimental import pallas as pl
from jax.experimental.pallas import tpu as pltpu

_B, _S, _HID = 2, 2048, 2048
_INTER = 5504
_RET = 128
_NE = 64
_TOPK = 8
_NK = 8

_INTER_PAD = 5632
_TN = 1024
_TK = 512


def _route_proj_kernel(h0_ref, h1_ref, wq_ref, keys_ref, r0_ref, r1_ref):
    dn = (((0,), (0,)), ((), ()))
    wq = wq_ref[...]
    k0 = keys_ref[0:64, :]
    k1 = keys_ref[64:128, :]
    p0a = jax.lax.dot_general(wq[0:64, :], k0, dn, preferred_element_type=jnp.float32)
    p0b = jax.lax.dot_general(wq[64:128, :], k0, dn, preferred_element_type=jnp.float32)
    p1a = jax.lax.dot_general(wq[0:64, :], k1, dn, preferred_element_type=jnp.float32)
    p1b = jax.lax.dot_general(wq[64:128, :], k1, dn, preferred_element_type=jnp.float32)
    P0 = jnp.concatenate([p0a, p0b], axis=1)
    P1 = jnp.concatenate([p1a, p1b], axis=1)
    r0_ref[...] = jnp.dot(h0_ref[...], P0, preferred_element_type=jnp.float32)
    r1_ref[...] = jnp.dot(h1_ref[...], P1, preferred_element_type=jnp.float32)


def _moe_mlp_kernel(rw0_ref, rw1_ref, h_ref, down_ref, up_ref,
                    wg_ref, wu_ref, wd_ref, out_ref, w64_ref,
                    a_even_ref, a_odd_ref):
    k = pl.program_id(1)
    kt = pl.num_programs(1) - 1

    @pl.when(k == 0)
    def _routing():
        rw0 = rw0_ref[...]
        rw1 = rw1_ref[...]
        col = jax.lax.broadcasted_iota(jnp.int32, (8, 64), 1)
        row = jax.lax.broadcasted_iota(jnp.int32, (8, 64), 0)
        e1 = (col // 8 == row).astype(jnp.float32)
        e2 = (col % 8 == row).astype(jnp.float32)
        s64 = (jnp.dot(rw0, e1, preferred_element_type=jnp.float32)
               + jnp.dot(rw1, e2, preferred_element_type=jnp.float32))
        cur = s64
        m0 = jnp.max(cur, axis=1, keepdims=True)
        m = m0
        for _ in range(_TOPK - 1):
            cur = jnp.where(cur >= m, -jnp.inf, cur)
            m = jnp.max(cur, axis=1, keepdims=True)
        mask = s64 >= m
        p = jnp.where(mask, jnp.exp(s64 - m0), 0.0)
        p = p / jnp.sum(p, axis=1, keepdims=True)
        L = jnp.dot(h_ref[...], down_ref[...],
                    preferred_element_type=jnp.float32)
        w64_ref[...] = L * jax.nn.sigmoid(L) * p
        out_ref[...] = jnp.dot(w64_ref[...], up_ref[...],
                               preferred_element_type=jnp.float32)

    @pl.when(k < kt)
    def _gu():
        h = h_ref[...]
        g = jnp.dot(h, wg_ref[...], preferred_element_type=jnp.float32)
        u = jnp.dot(h, wu_ref[...], preferred_element_type=jnp.float32)
        a = (g * jax.nn.sigmoid(g) * u).astype(jnp.bfloat16)

        @pl.when(k % 2 == 0)
        def _():
            a_even_ref[...] = a

        @pl.when(k % 2 == 1)
        def _():
            a_odd_ref[...] = a

    @pl.when(k > 0)
    def _acc():
        @pl.when(k % 2 == 1)
        def _():
            out_ref[...] += jnp.dot(
                a_even_ref[...], wd_ref[...], preferred_element_type=jnp.float32)

        @pl.when(k % 2 == 0)
        def _():
            out_ref[...] += jnp.dot(
                a_odd_ref[...], wd_ref[...], preferred_element_type=jnp.float32)


def kernel(hidden_states, Wq, keys, down_embed, up_embed, Wg, Wu, Wd):
    b, s, h = hidden_states.shape
    N = b * s
    hflat = hidden_states.reshape(N, h)
    keys2 = keys.reshape(2 * (_RET // 2), _NK)

    r0, r1 = pl.pallas_call(
        _route_proj_kernel,
        grid=(2,),
        in_specs=[
            pl.BlockSpec((s // 2, h), lambda i: (i, 0)),
            pl.BlockSpec((s // 2, h), lambda i: (i, 0)),
            pl.BlockSpec((_RET, h), lambda i: (0, 0)),
            pl.BlockSpec((2 * (_RET // 2), _NK), lambda i: (0, 0)),
        ],
        out_specs=[
            pl.BlockSpec((s // 2, 16), lambda i: (i, 0)),
            pl.BlockSpec((s // 2, 16), lambda i: (i, 0)),
        ],
        out_shape=[
            jax.ShapeDtypeStruct((s, 16), jnp.float32),
            jax.ShapeDtypeStruct((s, 16), jnp.float32),
        ],
    )(hidden_states[0], hidden_states[1], Wq, keys2)

    rw0 = r0.reshape(N, _NK)
    rw1 = r1.reshape(N, _NK)

    hflat_b = hflat.astype(jnp.bfloat16)
    down_b = down_embed.T.astype(jnp.bfloat16)
    wg_p = jnp.pad(Wg.T.astype(jnp.bfloat16), ((0, 0), (0, _INTER_PAD - _INTER)))
    wu_p = jnp.pad(Wu.T.astype(jnp.bfloat16), ((0, 0), (0, _INTER_PAD - _INTER)))
    wd_p = jnp.pad(Wd.T.astype(jnp.bfloat16), ((0, _INTER_PAD - _INTER), (0, 0)))

    nt = N // _TN
    kt = _INTER_PAD // _TK
    out = pl.pallas_call(
        _moe_mlp_kernel,
        grid=(nt, kt + 1),
        in_specs=[
            pl.BlockSpec((_TN, _NK), lambda n, k: (n, 0)),
            pl.BlockSpec((_TN, _NK), lambda n, k: (n, 0)),
            pl.BlockSpec((_TN, h), lambda n, k: (n, 0)),
            pl.BlockSpec((h, _NE), lambda n, k: (0, 0)),
            pl.BlockSpec((_NE, h), lambda n, k: (0, 0)),
            pl.BlockSpec((h, _TK), lambda n, k: (0, jnp.minimum(k, kt - 1))),
            pl.BlockSpec((h, _TK), lambda n, k: (0, jnp.minimum(k, kt - 1))),
            pl.BlockSpec((_TK, h), lambda n, k: (jnp.maximum(k - 1, 0), 0)),
        ],
        out_specs=pl.BlockSpec((_TN, h), lambda n, k: (n, 0)),
        out_shape=jax.ShapeDtypeStruct((N, h), jnp.float32),
        scratch_shapes=[pltpu.VMEM((_TN, _NE), jnp.float32),
                        pltpu.VMEM((_TN, _TK), jnp.bfloat16),
                        pltpu.VMEM((_TN, _TK), jnp.bfloat16)],
        compiler_params=pltpu.CompilerParams(
            dimension_semantics=("arbitrary", "arbitrary"),
        ),
    )(rw0, rw1, hflat_b, down_b, up_embed, wg_p, wu_p, wd_p)

    return out.reshape(b, s, h)


# speedup vs baseline: 1.0109x; 1.0109x over previous
import jax
import jax.numpy as jnp
from jax.experimental import pallas as pl
from jax.experimental.pallas import tpu as pltpu

_B, _S, _HID = 2, 2048, 2048
_INTER = 5504
_RET = 128
_NE = 64
_TOPK = 8
_NK = 8

_INTER_PAD = 5632
_TN = 512
_TK = 512


def _moe_mlp_kernel(h0_ref, h1_ref, hflat_ref, wq_ref, keys_ref,
                    down_ref, up_ref, wg_ref, wu_ref, wd_ref, out_ref,
                    hb_ref, a_even_ref, a_odd_ref):
    k = pl.program_id(1)
    kt = pl.num_programs(1) - 1
    f32 = jnp.float32

    @pl.when(k == 0)
    def _routing():
        hb = hflat_ref[...].astype(jnp.bfloat16)
        hb_ref[...] = hb

        dn0 = (((0,), (0,)), ((), ()))
        wq = wq_ref[...]
        k0 = keys_ref[0:64, :]
        k1 = keys_ref[64:128, :]
        p0a = jax.lax.dot_general(wq[0:64, :], k0, dn0, preferred_element_type=f32)
        p0b = jax.lax.dot_general(wq[64:128, :], k0, dn0, preferred_element_type=f32)
        p1a = jax.lax.dot_general(wq[0:64, :], k1, dn0, preferred_element_type=f32)
        p1b = jax.lax.dot_general(wq[64:128, :], k1, dn0, preferred_element_type=f32)
        h0 = h0_ref[0]
        h1 = h1_ref[0]
        a0a = jnp.dot(h0, p0a, preferred_element_type=f32)
        a0b = jnp.dot(h0, p0b, preferred_element_type=f32)
        a1a = jnp.dot(h1, p1a, preferred_element_type=f32)
        a1b = jnp.dot(h1, p1b, preferred_element_type=f32)

        col = jax.lax.broadcasted_iota(jnp.int32, (8, 64), 1)
        row = jax.lax.broadcasted_iota(jnp.int32, (8, 64), 0)
        e1 = (col // 8 == row).astype(f32)
        e2 = (col % 8 == row).astype(f32)

        def _masked_softmax_top8(s64):
            cur = s64
            m0 = jnp.max(cur, axis=1, keepdims=True)
            m = m0
            for _ in range(_TOPK - 1):
                cur = jnp.where(cur >= m, -jnp.inf, cur)
                m = jnp.max(cur, axis=1, keepdims=True)
            p = jnp.where(s64 >= m, jnp.exp(s64 - m0), 0.0)
            return p / jnp.sum(p, axis=1, keepdims=True)

        s64e = (jnp.dot(a0a, e1, preferred_element_type=f32)
                + jnp.dot(a1a, e2, preferred_element_type=f32))
        s64o = (jnp.dot(a0b, e1, preferred_element_type=f32)
                + jnp.dot(a1b, e2, preferred_element_type=f32))
        pe = _masked_softmax_top8(s64e)
        po = _masked_softmax_top8(s64o)

        rr = jax.lax.broadcasted_iota(jnp.int32, (_TN, _TN // 2), 0)
        cc = jax.lax.broadcasted_iota(jnp.int32, (_TN, _TN // 2), 1)
        ea = (rr == 2 * cc).astype(f32)
        eb = (rr == 2 * cc + 1).astype(f32)
        p = (jnp.dot(ea, pe, preferred_element_type=f32)
             + jnp.dot(eb, po, preferred_element_type=f32))

        L = jax.lax.dot_general(hb, down_ref[...].astype(jnp.bfloat16),
                                (((1,), (1,)), ((), ())),
                                preferred_element_type=f32)
        w64 = L * jax.nn.sigmoid(L) * p
        out_ref[...] = jnp.dot(w64, up_ref[...], preferred_element_type=f32)

    dnT = (((1,), (1,)), ((), ()))

    @pl.when(k < kt)
    def _gu():
        hb = hb_ref[...]
        g = jax.lax.dot_general(hb, wg_ref[...], dnT, preferred_element_type=f32)
        u = jax.lax.dot_general(hb, wu_ref[...], dnT, preferred_element_type=f32)
        a = (g * jax.nn.sigmoid(g) * u).astype(jnp.bfloat16)

        @pl.when(k % 2 == 0)
        def _():
            a_even_ref[...] = a

        @pl.when(k % 2 == 1)
        def _():
            a_odd_ref[...] = a

    @pl.when(k > 0)
    def _acc():
        @pl.when(k % 2 == 1)
        def _():
            out_ref[...] += jax.lax.dot_general(
                a_even_ref[...], wd_ref[...], dnT, preferred_element_type=f32)

        @pl.when(k % 2 == 0)
        def _():
            out_ref[...] += jax.lax.dot_general(
                a_odd_ref[...], wd_ref[...], dnT, preferred_element_type=f32)


def kernel(hidden_states, Wq, keys, down_embed, up_embed, Wg, Wu, Wd):
    b, s, h = hidden_states.shape
    N = b * s
    hflat = hidden_states.reshape(N, h)
    keys2 = keys.reshape(2 * (_RET // 2), _NK)

    wg_p = jnp.pad(Wg.astype(jnp.bfloat16), ((0, _INTER_PAD - _INTER), (0, 0)))
    wu_p = jnp.pad(Wu.astype(jnp.bfloat16), ((0, _INTER_PAD - _INTER), (0, 0)))
    wd_p = jnp.pad(Wd.astype(jnp.bfloat16), ((0, 0), (0, _INTER_PAD - _INTER)))

    nt = N // _TN
    kt = _INTER_PAD // _TK
    out = pl.pallas_call(
        _moe_mlp_kernel,
        grid=(nt, kt + 1),
        in_specs=[
            pl.BlockSpec((1, _TN // 2, h), lambda n, k: (0, n, 0)),
            pl.BlockSpec((1, _TN // 2, h), lambda n, k: (1, n, 0)),
            pl.BlockSpec((_TN, h), lambda n, k: (n, 0)),
            pl.BlockSpec((_RET, h), lambda n, k: (0, 0)),
            pl.BlockSpec((2 * (_RET // 2), _NK), lambda n, k: (0, 0)),
            pl.BlockSpec((_NE, h), lambda n, k: (0, 0)),
            pl.BlockSpec((_NE, h), lambda n, k: (0, 0)),
            pl.BlockSpec((_TK, h), lambda n, k: (jnp.minimum(k, kt - 1), 0)),
            pl.BlockSpec((_TK, h), lambda n, k: (jnp.minimum(k, kt - 1), 0)),
            pl.BlockSpec((h, _TK), lambda n, k: (0, jnp.maximum(k - 1, 0))),
        ],
        out_specs=pl.BlockSpec((_TN, h), lambda n, k: (n, 0)),
        out_shape=jax.ShapeDtypeStruct((N, h), jnp.float32),
        scratch_shapes=[pltpu.VMEM((_TN, h), jnp.bfloat16),
                        pltpu.VMEM((_TN, _TK), jnp.bfloat16),
                        pltpu.VMEM((_TN, _TK), jnp.bfloat16)],
        compiler_params=pltpu.CompilerParams(
            dimension_semantics=("arbitrary", "arbitrary"),
        ),
    )(hidden_states, hidden_states, hflat, Wq, keys2,
      down_embed, up_embed, wg_p, wu_p, wd_p)

    return out.reshape(b, s, h)


# speedup vs baseline: 1.1095x; 1.0975x over previous
import jax
import jax.numpy as jnp
from jax.experimental import pallas as pl
from jax.experimental.pallas import tpu as pltpu

_B, _S, _HID = 2, 2048, 2048
_INTER = 5504
_RET = 128
_NE = 64
_TOPK = 8
_NK = 8

_INTER_PAD = 5632
_TN = 1024
_TK = 512


def _moe_mlp_kernel(h0_ref, h1_ref, hflat_ref, wq_ref, keys_ref,
                    down_ref, up_ref, wg_ref, wu_ref, wd_ref, out_ref,
                    a_even_ref, a_odd_ref):
    k = pl.program_id(1)
    kt = pl.num_programs(1) - 1
    f32 = jnp.float32

    @pl.when(k == 0)
    def _routing():
        hb = hflat_ref[...]

        dn0 = (((0,), (0,)), ((), ()))
        wq = wq_ref[...]
        k0 = keys_ref[0:64, :]
        k1 = keys_ref[64:128, :]
        p0a = jax.lax.dot_general(wq[0:64, :], k0, dn0, preferred_element_type=f32)
        p0b = jax.lax.dot_general(wq[64:128, :], k0, dn0, preferred_element_type=f32)
        p1a = jax.lax.dot_general(wq[0:64, :], k1, dn0, preferred_element_type=f32)
        p1b = jax.lax.dot_general(wq[64:128, :], k1, dn0, preferred_element_type=f32)
        h0 = h0_ref[...]
        h1 = h1_ref[...]
        a0a = jnp.dot(h0, p0a, preferred_element_type=f32)
        a0b = jnp.dot(h0, p0b, preferred_element_type=f32)
        a1a = jnp.dot(h1, p1a, preferred_element_type=f32)
        a1b = jnp.dot(h1, p1b, preferred_element_type=f32)

        col = jax.lax.broadcasted_iota(jnp.int32, (8, 64), 1)
        row = jax.lax.broadcasted_iota(jnp.int32, (8, 64), 0)
        e1 = (col // 8 == row).astype(f32)
        e2 = (col % 8 == row).astype(f32)

        def _masked_softmax_top8(s64):
            cur = s64
            m0 = jnp.max(cur, axis=1, keepdims=True)
            m = m0
            for _ in range(_TOPK - 1):
                cur = jnp.where(cur >= m, -jnp.inf, cur)
                m = jnp.max(cur, axis=1, keepdims=True)
            p = jnp.where(s64 >= m, jnp.exp(s64 - m0), 0.0)
            return p / jnp.sum(p, axis=1, keepdims=True)

        s64e = (jnp.dot(a0a, e1, preferred_element_type=f32)
                + jnp.dot(a1a, e2, preferred_element_type=f32))
        s64o = (jnp.dot(a0b, e1, preferred_element_type=f32)
                + jnp.dot(a1b, e2, preferred_element_type=f32))
        pe = _masked_softmax_top8(s64e)
        po = _masked_softmax_top8(s64o)

        rr = jax.lax.broadcasted_iota(jnp.int32, (_TN, _TN // 2), 0)
        cc = jax.lax.broadcasted_iota(jnp.int32, (_TN, _TN // 2), 1)
        ea = (rr == 2 * cc).astype(f32)
        eb = (rr == 2 * cc + 1).astype(f32)
        p = (jnp.dot(ea, pe, preferred_element_type=f32)
             + jnp.dot(eb, po, preferred_element_type=f32))

        L = jax.lax.dot_general(hb, down_ref[...].astype(jnp.bfloat16),
                                (((1,), (1,)), ((), ())),
                                preferred_element_type=f32)
        w64 = L * jax.nn.sigmoid(L) * p
        out_ref[...] = jnp.dot(w64, up_ref[...], preferred_element_type=f32)

    dnT = (((1,), (1,)), ((), ()))

    @pl.when(k < kt)
    def _gu():
        hb = hflat_ref[...]
        g = jax.lax.dot_general(hb, wg_ref[...], dnT, preferred_element_type=f32)
        u = jax.lax.dot_general(hb, wu_ref[...], dnT, preferred_element_type=f32)
        a = (g * jax.nn.sigmoid(g) * u).astype(jnp.bfloat16)

        @pl.when(k % 2 == 0)
        def _():
            a_even_ref[...] = a

        @pl.when(k % 2 == 1)
        def _():
            a_odd_ref[...] = a

    @pl.when(k > 0)
    def _acc():
        @pl.when(k % 2 == 1)
        def _():
            out_ref[...] += jax.lax.dot_general(
                a_even_ref[...], wd_ref[...], dnT, preferred_element_type=f32)

        @pl.when(k % 2 == 0)
        def _():
            out_ref[...] += jax.lax.dot_general(
                a_odd_ref[...], wd_ref[...], dnT, preferred_element_type=f32)


def kernel(hidden_states, Wq, keys, down_embed, up_embed, Wg, Wu, Wd):
    b, s, h = hidden_states.shape
    N = b * s
    hflat = hidden_states.reshape(N, h).astype(jnp.bfloat16)
    keys2 = keys.reshape(2 * (_RET // 2), _NK)

    wg_p = jnp.pad(Wg.astype(jnp.bfloat16), ((0, _INTER_PAD - _INTER), (0, 0)))
    wu_p = jnp.pad(Wu.astype(jnp.bfloat16), ((0, _INTER_PAD - _INTER), (0, 0)))
    wd_p = jnp.pad(Wd.astype(jnp.bfloat16), ((0, 0), (0, _INTER_PAD - _INTER)))

    nt = N // _TN
    kt = _INTER_PAD // _TK
    out = pl.pallas_call(
        _moe_mlp_kernel,
        grid=(nt, kt + 1),
        in_specs=[
            pl.BlockSpec((_TN // 2, h), lambda n, k: (n, 0)),
            pl.BlockSpec((_TN // 2, h), lambda n, k: (n + _S // (_TN // 2), 0)),
            pl.BlockSpec((_TN, h), lambda n, k: (n, 0)),
            pl.BlockSpec((_RET, h), lambda n, k: (0, 0)),
            pl.BlockSpec((2 * (_RET // 2), _NK), lambda n, k: (0, 0)),
            pl.BlockSpec((_NE, h), lambda n, k: (0, 0)),
            pl.BlockSpec((_NE, h), lambda n, k: (0, 0)),
            pl.BlockSpec((_TK, h), lambda n, k: (jnp.minimum(k, kt - 1), 0)),
            pl.BlockSpec((_TK, h), lambda n, k: (jnp.minimum(k, kt - 1), 0)),
            pl.BlockSpec((h, _TK), lambda n, k: (0, jnp.maximum(k - 1, 0))),
        ],
        out_specs=pl.BlockSpec((_TN, h), lambda n, k: (n, 0)),
        out_shape=jax.ShapeDtypeStruct((N, h), jnp.float32),
        scratch_shapes=[pltpu.VMEM((_TN, _TK), jnp.bfloat16),
                        pltpu.VMEM((_TN, _TK), jnp.bfloat16)],
        compiler_params=pltpu.CompilerParams(
            dimension_semantics=("arbitrary", "arbitrary"),
        ),
    )(hflat, hflat, hflat, Wq, keys2,
      down_embed, up_embed, wg_p, wu_p, wd_p)

    return out.reshape(b, s, h)


# speedup vs baseline: 1.2535x; 1.1298x over previous
import jax
import jax.numpy as jnp
from jax.experimental import pallas as pl
from jax.experimental.pallas import tpu as pltpu

_B, _S, _HID = 2, 2048, 2048
_INTER = 5504
_RET = 128
_NE = 64
_TOPK = 8
_NK = 8

_INTER_PAD = 5632
_TN = 1024
_TK = 512


def _repack_kernel(wg_ref, wu_ref, wd_ref, h_ref,
                   wgo_ref, wuo_ref, wdo_ref, ho_ref):
    i = pl.program_id(0)
    nchunk = pl.num_programs(0)
    wg = wg_ref[...].astype(jnp.bfloat16)
    wu = wu_ref[...].astype(jnp.bfloat16)
    wd = wd_ref[...].astype(jnp.bfloat16)

    @pl.when(i < nchunk - 1)
    def _():
        wgo_ref[...] = wg
        wuo_ref[...] = wu
        wdo_ref[...] = wd

    @pl.when(i == nchunk - 1)
    def _():
        valid = _INTER % wg.shape[0]
        rr = jax.lax.broadcasted_iota(jnp.int32, wg.shape, 0)
        zb = jnp.zeros((), jnp.bfloat16)
        wgo_ref[...] = jnp.where(rr < valid, wg, zb)
        wuo_ref[...] = jnp.where(rr < valid, wu, zb)
        cc = jax.lax.broadcasted_iota(jnp.int32, wd.shape, 1)
        wdo_ref[...] = jnp.where(cc < valid, wd, zb)

    ho_ref[...] = h_ref[...].astype(jnp.bfloat16)


def _moe_mlp_kernel(h0_ref, h1_ref, hflat_ref, wq_ref, keys_ref,
                    down_ref, up_ref, wg_ref, wu_ref, wd_ref, out_ref,
                    a_even_ref, a_odd_ref):
    k = pl.program_id(1)
    kt = pl.num_programs(1) - 1
    f32 = jnp.float32

    @pl.when(k == 0)
    def _routing():
        hb = hflat_ref[...]

        dn0 = (((0,), (0,)), ((), ()))
        wq = wq_ref[...]
        k0 = keys_ref[0:64, :]
        k1 = keys_ref[64:128, :]
        p0a = jax.lax.dot_general(wq[0:64, :], k0, dn0, preferred_element_type=f32)
        p0b = jax.lax.dot_general(wq[64:128, :], k0, dn0, preferred_element_type=f32)
        p1a = jax.lax.dot_general(wq[0:64, :], k1, dn0, preferred_element_type=f32)
        p1b = jax.lax.dot_general(wq[64:128, :], k1, dn0, preferred_element_type=f32)
        h0 = h0_ref[...]
        h1 = h1_ref[...]
        a0a = jnp.dot(h0, p0a, preferred_element_type=f32)
        a0b = jnp.dot(h0, p0b, preferred_element_type=f32)
        a1a = jnp.dot(h1, p1a, preferred_element_type=f32)
        a1b = jnp.dot(h1, p1b, preferred_element_type=f32)

        col = jax.lax.broadcasted_iota(jnp.int32, (8, 64), 1)
        row = jax.lax.broadcasted_iota(jnp.int32, (8, 64), 0)
        e1 = (col // 8 == row).astype(f32)
        e2 = (col % 8 == row).astype(f32)

        def _masked_softmax_top8(s64):
            cur = s64
            m0 = jnp.max(cur, axis=1, keepdims=True)
            m = m0
            for _ in range(_TOPK - 1):
                cur = jnp.where(cur >= m, -jnp.inf, cur)
                m = jnp.max(cur, axis=1, keepdims=True)
            p = jnp.where(s64 >= m, jnp.exp(s64 - m0), 0.0)
            return p / jnp.sum(p, axis=1, keepdims=True)

        s64e = (jnp.dot(a0a, e1, preferred_element_type=f32)
                + jnp.dot(a1a, e2, preferred_element_type=f32))
        s64o = (jnp.dot(a0b, e1, preferred_element_type=f32)
                + jnp.dot(a1b, e2, preferred_element_type=f32))
        pe = _masked_softmax_top8(s64e)
        po = _masked_softmax_top8(s64o)

        rr = jax.lax.broadcasted_iota(jnp.int32, (_TN, _TN // 2), 0)
        cc = jax.lax.broadcasted_iota(jnp.int32, (_TN, _TN // 2), 1)
        ea = (rr == 2 * cc).astype(f32)
        eb = (rr == 2 * cc + 1).astype(f32)
        p = (jnp.dot(ea, pe, preferred_element_type=f32)
             + jnp.dot(eb, po, preferred_element_type=f32))

        L = jax.lax.dot_general(hb, down_ref[...].astype(jnp.bfloat16),
                                (((1,), (1,)), ((), ())),
                                preferred_element_type=f32)
        w64 = L * jax.nn.sigmoid(L) * p
        out_ref[...] = jnp.dot(w64, up_ref[...], preferred_element_type=f32)

    dnT = (((1,), (1,)), ((), ()))

    @pl.when(k < kt)
    def _gu():
        hb = hflat_ref[...]
        g = jax.lax.dot_general(hb, wg_ref[...], dnT, preferred_element_type=f32)
        u = jax.lax.dot_general(hb, wu_ref[...], dnT, preferred_element_type=f32)
        a = (g * jax.nn.sigmoid(g) * u).astype(jnp.bfloat16)

        @pl.when(k % 2 == 0)
        def _():
            a_even_ref[...] = a

        @pl.when(k % 2 == 1)
        def _():
            a_odd_ref[...] = a

    @pl.when(k > 0)
    def _acc():
        @pl.when(k % 2 == 1)
        def _():
            out_ref[...] += jax.lax.dot_general(
                a_even_ref[...], wd_ref[...], dnT, preferred_element_type=f32)

        @pl.when(k % 2 == 0)
        def _():
            out_ref[...] += jax.lax.dot_general(
                a_odd_ref[...], wd_ref[...], dnT, preferred_element_type=f32)


def kernel(hidden_states, Wq, keys, down_embed, up_embed, Wg, Wu, Wd):
    b, s, h = hidden_states.shape
    N = b * s
    hflat_f = hidden_states.reshape(N, h)
    keys2 = keys.reshape(2 * (_RET // 2), _NK)

    rp = 256
    nchunk = _INTER_PAD // rp
    wg_p, wu_p, wd_p, hflat = pl.pallas_call(
        _repack_kernel,
        grid=(nchunk,),
        in_specs=[
            pl.BlockSpec((rp, h), lambda i: (i, 0)),
            pl.BlockSpec((rp, h), lambda i: (i, 0)),
            pl.BlockSpec((h, rp), lambda i: (0, i)),
            pl.BlockSpec((rp, h), lambda i: (jnp.minimum(i, N // rp - 1), 0)),
        ],
        out_specs=[
            pl.BlockSpec((rp, h), lambda i: (i, 0)),
            pl.BlockSpec((rp, h), lambda i: (i, 0)),
            pl.BlockSpec((h, rp), lambda i: (0, i)),
            pl.BlockSpec((rp, h), lambda i: (jnp.minimum(i, N // rp - 1), 0)),
        ],
        out_shape=[
            jax.ShapeDtypeStruct((_INTER_PAD, h), jnp.bfloat16),
            jax.ShapeDtypeStruct((_INTER_PAD, h), jnp.bfloat16),
            jax.ShapeDtypeStruct((h, _INTER_PAD), jnp.bfloat16),
            jax.ShapeDtypeStruct((N, h), jnp.bfloat16),
        ],
    )(Wg, Wu, Wd, hflat_f)

    nt = N // _TN
    kt = _INTER_PAD // _TK
    out = pl.pallas_call(
        _moe_mlp_kernel,
        grid=(nt, kt + 1),
        in_specs=[
            pl.BlockSpec((_TN // 2, h), lambda n, k: (n, 0)),
            pl.BlockSpec((_TN // 2, h), lambda n, k: (n + _S // (_TN // 2), 0)),
            pl.BlockSpec((_TN, h), lambda n, k: (n, 0)),
            pl.BlockSpec((_RET, h), lambda n, k: (0, 0)),
            pl.BlockSpec((2 * (_RET // 2), _NK), lambda n, k: (0, 0)),
            pl.BlockSpec((_NE, h), lambda n, k: (0, 0)),
            pl.BlockSpec((_NE, h), lambda n, k: (0, 0)),
            pl.BlockSpec((_TK, h), lambda n, k: (jnp.minimum(k, kt - 1), 0)),
            pl.BlockSpec((_TK, h), lambda n, k: (jnp.minimum(k, kt - 1), 0)),
            pl.BlockSpec((h, _TK), lambda n, k: (0, jnp.maximum(k - 1, 0))),
        ],
        out_specs=pl.BlockSpec((_TN, h), lambda n, k: (n, 0)),
        out_shape=jax.ShapeDtypeStruct((N, h), jnp.float32),
        scratch_shapes=[pltpu.VMEM((_TN, _TK), jnp.bfloat16),
                        pltpu.VMEM((_TN, _TK), jnp.bfloat16)],
        compiler_params=pltpu.CompilerParams(
            dimension_semantics=("arbitrary", "arbitrary"),
        ),
    )(hflat, hflat, hflat, Wq, keys2,
      down_embed, up_embed, wg_p, wu_p, wd_p)

    return out.reshape(b, s, h)


# speedup vs baseline: 1.3078x; 1.0434x over previous
import jax
import jax.numpy as jnp
from jax.experimental import pallas as pl
from jax.experimental.pallas import tpu as pltpu

_B, _S, _HID = 2, 2048, 2048
_INTER = 5504
_RET = 128
_NE = 64
_TOPK = 8
_NK = 8

_INTER_PAD = 5632
_TN = 1024
_TK = 512


def _repack_kernel(wg_ref, wu_ref, wd_ref, h_ref,
                   wgo_ref, wuo_ref, wdo_ref, ho_ref):
    i = pl.program_id(0)
    nchunk = pl.num_programs(0)
    wg = wg_ref[...].astype(jnp.bfloat16)
    wu = wu_ref[...].astype(jnp.bfloat16)
    wd = wd_ref[...].astype(jnp.bfloat16)

    @pl.when(i < nchunk - 1)
    def _():
        wgo_ref[...] = wg
        wuo_ref[...] = wu
        wdo_ref[...] = wd

    @pl.when(i == nchunk - 1)
    def _():
        valid = _INTER % wg.shape[0]
        rr = jax.lax.broadcasted_iota(jnp.int32, wg.shape, 0)
        zb = jnp.zeros((), jnp.bfloat16)
        wgo_ref[...] = jnp.where(rr < valid, wg, zb)
        wuo_ref[...] = jnp.where(rr < valid, wu, zb)
        cc = jax.lax.broadcasted_iota(jnp.int32, wd.shape, 1)
        wdo_ref[...] = jnp.where(cc < valid, wd, zb)

    ho_ref[...] = h_ref[...].astype(jnp.bfloat16)


def _moe_mlp_kernel(h0_ref, h1_ref, hflat_ref, wq_ref, keys_ref,
                    down_ref, up_ref, wg_ref, wu_ref, wd_ref, out_ref,
                    a_ref):
    k = pl.program_id(1)
    kt = pl.num_programs(1) - 1
    f32 = jnp.float32
    dnT = (((1,), (1,)), ((), ()))

    @pl.when(k == 0)
    def _routing():
        hb = hflat_ref[...]

        dn0 = (((0,), (0,)), ((), ()))
        wq = wq_ref[...]
        k0 = keys_ref[0:64, :]
        k1 = keys_ref[64:128, :]
        p0a = jax.lax.dot_general(wq[0:64, :], k0, dn0, preferred_element_type=f32)
        p0b = jax.lax.dot_general(wq[64:128, :], k0, dn0, preferred_element_type=f32)
        p1a = jax.lax.dot_general(wq[0:64, :], k1, dn0, preferred_element_type=f32)
        p1b = jax.lax.dot_general(wq[64:128, :], k1, dn0, preferred_element_type=f32)
        h0 = h0_ref[...]
        h1 = h1_ref[...]
        a0a = jnp.dot(h0, p0a, preferred_element_type=f32)
        a0b = jnp.dot(h0, p0b, preferred_element_type=f32)
        a1a = jnp.dot(h1, p1a, preferred_element_type=f32)
        a1b = jnp.dot(h1, p1b, preferred_element_type=f32)

        col = jax.lax.broadcasted_iota(jnp.int32, (8, 64), 1)
        row = jax.lax.broadcasted_iota(jnp.int32, (8, 64), 0)
        e1 = (col // 8 == row).astype(f32)
        e2 = (col % 8 == row).astype(f32)

        def _masked_softmax_top8(s64):
            cur = s64
            m0 = jnp.max(cur, axis=1, keepdims=True)
            m = m0
            for _ in range(_TOPK - 1):
                cur = jnp.where(cur >= m, -jnp.inf, cur)
                m = jnp.max(cur, axis=1, keepdims=True)
            p = jnp.where(s64 >= m, jnp.exp(s64 - m0), 0.0)
            return p / jnp.sum(p, axis=1, keepdims=True)

        s64e = (jnp.dot(a0a, e1, preferred_element_type=f32)
                + jnp.dot(a1a, e2, preferred_element_type=f32))
        s64o = (jnp.dot(a0b, e1, preferred_element_type=f32)
                + jnp.dot(a1b, e2, preferred_element_type=f32))
        pe = _masked_softmax_top8(s64e)
        po = _masked_softmax_top8(s64o)

        rr = jax.lax.broadcasted_iota(jnp.int32, (_TN, _TN // 2), 0)
        cc = jax.lax.broadcasted_iota(jnp.int32, (_TN, _TN // 2), 1)
        ea = (rr == 2 * cc).astype(f32)
        eb = (rr == 2 * cc + 1).astype(f32)
        p = (jnp.dot(ea, pe, preferred_element_type=f32)
             + jnp.dot(eb, po, preferred_element_type=f32))

        L = jax.lax.dot_general(hb, down_ref[...].astype(jnp.bfloat16),
                                (((1,), (1,)), ((), ())),
                                preferred_element_type=f32)
        w64 = L * jax.nn.sigmoid(L) * p
        out_ref[...] = jnp.dot(w64, up_ref[...], preferred_element_type=f32)

        g = jax.lax.dot_general(hb, wg_ref[...], dnT, preferred_element_type=f32)
        u = jax.lax.dot_general(hb, wu_ref[...], dnT, preferred_element_type=f32)
        a_ref[pl.ds(0, 1)] = (g * jax.nn.sigmoid(g) * u).astype(jnp.bfloat16)[None]

    @pl.when(jnp.logical_and(k > 0, k < kt))
    def _steady():
        hb = hflat_ref[...]
        rd = jax.lax.rem(k + 1, 2)
        wr = jax.lax.rem(k, 2)
        ap = a_ref[pl.ds(rd, 1)][0]
        out_ref[...] += jax.lax.dot_general(
            ap, wd_ref[...], dnT, preferred_element_type=f32)
        g = jax.lax.dot_general(hb, wg_ref[...], dnT, preferred_element_type=f32)
        u = jax.lax.dot_general(hb, wu_ref[...], dnT, preferred_element_type=f32)
        a_ref[pl.ds(wr, 1)] = (g * jax.nn.sigmoid(g) * u).astype(jnp.bfloat16)[None]

    @pl.when(k == kt)
    def _drain():
        out_ref[...] += jax.lax.dot_general(
            a_ref[pl.ds((kt - 1) % 2, 1)][0], wd_ref[...], dnT,
            preferred_element_type=f32)


def kernel(hidden_states, Wq, keys, down_embed, up_embed, Wg, Wu, Wd):
    b, s, h = hidden_states.shape
    N = b * s
    hflat_f = hidden_states.reshape(N, h)
    keys2 = keys.reshape(2 * (_RET // 2), _NK)

    rp = 256
    nchunk = _INTER_PAD // rp
    wg_p, wu_p, wd_p, hflat = pl.pallas_call(
        _repack_kernel,
        grid=(nchunk,),
        in_specs=[
            pl.BlockSpec((rp, h), lambda i: (i, 0)),
            pl.BlockSpec((rp, h), lambda i: (i, 0)),
            pl.BlockSpec((h, rp), lambda i: (0, i)),
            pl.BlockSpec((rp, h), lambda i: (jnp.minimum(i, N // rp - 1), 0)),
        ],
        out_specs=[
            pl.BlockSpec((rp, h), lambda i: (i, 0)),
            pl.BlockSpec((rp, h), lambda i: (i, 0)),
            pl.BlockSpec((h, rp), lambda i: (0, i)),
            pl.BlockSpec((rp, h), lambda i: (jnp.minimum(i, N // rp - 1), 0)),
        ],
        out_shape=[
            jax.ShapeDtypeStruct((_INTER_PAD, h), jnp.bfloat16),
            jax.ShapeDtypeStruct((_INTER_PAD, h), jnp.bfloat16),
            jax.ShapeDtypeStruct((h, _INTER_PAD), jnp.bfloat16),
            jax.ShapeDtypeStruct((N, h), jnp.bfloat16),
        ],
    )(Wg, Wu, Wd, hflat_f)

    nt = N // _TN
    kt = _INTER_PAD // _TK
    out = pl.pallas_call(
        _moe_mlp_kernel,
        grid=(nt, kt + 1),
        in_specs=[
            pl.BlockSpec((_TN // 2, h), lambda n, k: (n, 0)),
            pl.BlockSpec((_TN // 2, h), lambda n, k: (n + _S // (_TN // 2), 0)),
            pl.BlockSpec((_TN, h), lambda n, k: (n, 0)),
            pl.BlockSpec((_RET, h), lambda n, k: (0, 0)),
            pl.BlockSpec((2 * (_RET // 2), _NK), lambda n, k: (0, 0)),
            pl.BlockSpec((_NE, h), lambda n, k: (0, 0)),
            pl.BlockSpec((_NE, h), lambda n, k: (0, 0)),
            pl.BlockSpec((_TK, h), lambda n, k: (jnp.minimum(k, kt - 1), 0)),
            pl.BlockSpec((_TK, h), lambda n, k: (jnp.minimum(k, kt - 1), 0)),
            pl.BlockSpec((h, _TK), lambda n, k: (0, jnp.maximum(k - 1, 0))),
        ],
        out_specs=pl.BlockSpec((_TN, h), lambda n, k: (n, 0)),
        out_shape=jax.ShapeDtypeStruct((N, h), jnp.float32),
        scratch_shapes=[pltpu.VMEM((2, _TN, _TK), jnp.bfloat16)],
        compiler_params=pltpu.CompilerParams(
            dimension_semantics=("arbitrary", "arbitrary"),
        ),
    )(hflat, hflat, hflat, Wq, keys2,
      down_embed, up_embed, wg_p, wu_p, wd_p)

    return out.reshape(b, s, h)
